# baseline (speedup 1.0000x reference)
import jax
import jax.numpy as jnp
from jax.experimental import pallas as pl
from jax.experimental.pallas import tpu as pltpu

_K = 1024
_D = 64
_B = 16
_H = 32
_W = 32
_HW = _H * _W
_N = _B * _HW
_T = 2048
_PB = _T // _HW
_STEPS = _N // _T
_COMMIT = 0.25


def _vq_body(z_ref, zn_ref, e_ref, enc_ref, q_ref, loss_ref, perp_ref,
             cnt_ref, sq_ref, et_ref):
    s = pl.program_id(0)
    zt = z_ref[...]
    e = e_ref[...]

    @pl.when(s == 0)
    def _():
        et_ref[...] = e.T
    p = jax.lax.dot_general(zt, e, (((1,), (1,)), ((), ())),
                            preferred_element_type=jnp.float32)
    sz = jnp.sum(zt * zt, axis=1, keepdims=True)
    se = jnp.sum(e * e, axis=1)
    dist = (sz + se[None, :]) - 2.0 * p
    _C = _K // 128
    m = dist[:, 0:128]
    for c in range(1, _C):
        m = jnp.minimum(m, dist[:, 128 * c:128 * (c + 1)])
    dmin = jnp.min(m, axis=1, keepdims=True)
    cidx = jnp.full((_T, 128), _C - 1, jnp.int32)
    for c in range(_C - 1, -1, -1):
        cidx = jnp.where(dist[:, 128 * c:128 * (c + 1)] == m, c, cidx)
    lio = jax.lax.broadcasted_iota(jnp.int32, (_T, 128), 1)
    kcand = jnp.where(m == dmin, (cidx * 128 + lio).astype(jnp.float32),
                      float(2 * _K))
    idx = jnp.min(kcand, axis=1, keepdims=True).astype(jnp.int32)
    kio = jax.lax.broadcasted_iota(jnp.int32, (_T, _K), 1)
    onehot = (kio == idx).astype(jnp.float32)
    enc_ref[...] = onehot
    q_dt = jax.lax.dot_general(et_ref[...], onehot, (((1,), (1,)), ((), ())),
                               preferred_element_type=jnp.float32)
    z_dt = jnp.concatenate([zn_ref[i] for i in range(_PB)], axis=1)
    diff = q_dt - z_dt
    qst = z_dt + diff
    for i in range(_PB):
        q_ref[i] = qst[:, i * _HW:(i + 1) * _HW]
    bc = jax.lax.dot_general(jnp.ones((1, _T), jnp.float32), onehot,
                             (((1,), (0,)), ((), ())),
                             preferred_element_type=jnp.float32)
    bs = jnp.sum(jnp.sum(diff * diff, axis=1, keepdims=True),
                 axis=0, keepdims=True)


    @pl.when(s == 0)
    def _():
        cnt_ref[...] = bc
        sq_ref[...] = bs

    @pl.when(s > 0)
    def _():
        cnt_ref[...] += bc
        sq_ref[...] += bs

    @pl.when(s == _STEPS - 1)
    def _():
        avg = cnt_ref[...] * (1.0 / _N)
        ent = jnp.sum(avg * jnp.log(avg + 1e-10), axis=1, keepdims=True)
        perp_ref[...] = jnp.exp(-ent)
        loss_ref[...] = sq_ref[...] * (_COMMIT / (_N * _D))


def kernel(z_e, embedding_weight):
    z_flat = z_e.transpose(0, 2, 3, 1).reshape(_N, _D)
    z_nat = z_e.reshape(_B, _D, _HW)
    enc, q3, loss, perp = pl.pallas_call(
        _vq_body,
        grid=(_STEPS,),
        in_specs=[pl.BlockSpec((_T, _D), lambda s: (s, 0)),
                  pl.BlockSpec((_PB, _D, _HW), lambda s: (s, 0, 0)),
                  pl.BlockSpec((_K, _D), lambda s: (0, 0))],
        out_specs=[pl.BlockSpec((_T, _K), lambda s: (s, 0)),
                   pl.BlockSpec((_PB, _D, _HW), lambda s: (s, 0, 0)),
                   pl.BlockSpec((1, 1), lambda s: (0, 0)),
                   pl.BlockSpec((1, 1), lambda s: (0, 0))],
        out_shape=[jax.ShapeDtypeStruct((_N, _K), jnp.float32),
                   jax.ShapeDtypeStruct((_B, _D, _HW), jnp.float32),
                   jax.ShapeDtypeStruct((1, 1), jnp.float32),
                   jax.ShapeDtypeStruct((1, 1), jnp.float32)],
        scratch_shapes=[pltpu.VMEM((1, _K), jnp.float32),
                        pltpu.VMEM((1, 1), jnp.float32),
                        pltpu.VMEM((_D, _K), jnp.float32)],
        compiler_params=pltpu.CompilerParams(
            dimension_semantics=("arbitrary",)),
    )(z_flat, z_nat, embedding_weight)
    q_out = q3.reshape(_B, _D, _H, _W)
    return (q_out, loss[0, 0], perp[0, 0], enc)


# speedup vs baseline: 1.2715x; 1.2715x over previous
import jax
import jax.numpy as jnp
from jax.experimental import pallas as pl
from jax.experimental.pallas import tpu as pltpu

_K = 1024
_D = 64
_B = 16
_H = 32
_W = 32
_N = _B * _H * _W
_T = 2048
_STEPS = _N // _T
_COMMIT = 0.25


def _vq_body(z_ref, e_ref, enc_ref, q_ref, loss_ref, perp_ref, cnt_ref, sq_ref):
    s = pl.program_id(0)
    zt = z_ref[...]
    e = e_ref[...]
    p = jax.lax.dot_general(zt, e, (((1,), (1,)), ((), ())),
                            preferred_element_type=jnp.float32)
    sz = jnp.sum(zt * zt, axis=1, keepdims=True)
    se = jnp.sum(e * e, axis=1)
    dist = (sz + se[None, :]) - 2.0 * p
    _C = _K // 128
    m = dist[:, 0:128]
    for c in range(1, _C):
        m = jnp.minimum(m, dist[:, 128 * c:128 * (c + 1)])
    dmin = jnp.min(m, axis=1, keepdims=True)
    cidx = jnp.full((_T, 128), _C - 1, jnp.int32)
    for c in range(_C - 1, -1, -1):
        cidx = jnp.where(dist[:, 128 * c:128 * (c + 1)] == m, c, cidx)
    lio = jax.lax.broadcasted_iota(jnp.int32, (_T, 128), 1)
    kcand = jnp.where(m == dmin, (cidx * 128 + lio).astype(jnp.float32),
                      float(2 * _K))
    idx = jnp.min(kcand, axis=1, keepdims=True).astype(jnp.int32)
    kio = jax.lax.broadcasted_iota(jnp.int32, (_T, _K), 1)
    onehot = (kio == idx).astype(jnp.float32)
    enc_ref[...] = onehot
    q = jax.lax.dot_general(onehot, e, (((1,), (0,)), ((), ())),
                            preferred_element_type=jnp.float32)
    diff = q - zt
    q_ref[...] = zt + diff
    bc = jax.lax.dot_general(jnp.ones((1, _T), jnp.float32), onehot,
                             (((1,), (0,)), ((), ())),
                             preferred_element_type=jnp.float32)
    bs = jnp.sum(jnp.sum(diff * diff, axis=1, keepdims=True),
                 axis=0, keepdims=True)

    @pl.when(s == 0)
    def _():
        cnt_ref[...] = bc
        sq_ref[...] = bs

    @pl.when(s > 0)
    def _():
        cnt_ref[...] += bc
        sq_ref[...] += bs

    @pl.when(s == _STEPS - 1)
    def _():
        avg = cnt_ref[...] * (1.0 / _N)
        ent = jnp.sum(avg * jnp.log(avg + 1e-10), axis=1, keepdims=True)
        perp_ref[...] = jnp.exp(-ent)
        loss_ref[...] = sq_ref[...] * (_COMMIT / (_N * _D))


def kernel(z_e, embedding_weight):
    z_flat = z_e.transpose(0, 2, 3, 1).reshape(_N, _D)
    enc, qf, loss, perp = pl.pallas_call(
        _vq_body,
        grid=(_STEPS,),
        in_specs=[pl.BlockSpec((_T, _D), lambda s: (s, 0)),
                  pl.BlockSpec((_K, _D), lambda s: (0, 0))],
        out_specs=[pl.BlockSpec((_T, _K), lambda s: (s, 0)),
                   pl.BlockSpec((_T, _D), lambda s: (s, 0)),
                   pl.BlockSpec((1, 1), lambda s: (0, 0)),
                   pl.BlockSpec((1, 1), lambda s: (0, 0))],
        out_shape=[jax.ShapeDtypeStruct((_N, _K), jnp.float32),
                   jax.ShapeDtypeStruct((_N, _D), jnp.float32),
                   jax.ShapeDtypeStruct((1, 1), jnp.float32),
                   jax.ShapeDtypeStruct((1, 1), jnp.float32)],
        scratch_shapes=[pltpu.VMEM((1, _K), jnp.float32),
                        pltpu.VMEM((1, 1), jnp.float32)],
        compiler_params=pltpu.CompilerParams(
            dimension_semantics=("arbitrary",)),
    )(z_flat, embedding_weight)
    q_out = qf.reshape(_B, _H, _W, _D).transpose(0, 3, 1, 2)
    return (q_out, loss[0, 0], perp[0, 0], enc)
